# baseline (speedup 1.0000x reference)
import jax
import jax.numpy as jnp
from jax.experimental import pallas as pl
from jax.experimental.pallas import tpu as pltpu

_B = 256
_T = 52
_IN = 52
_NA = 13 * 4 * 13 * 2
_H = _NA * 2
_HAND = 13 * 4 * 13

_F32 = jnp.float32
_BF16 = jnp.bfloat16


def _gru_body(seq_ref, wih_r_ref, wih_z_ref, wih_n_ref,
              whh_r_ref, whh_z_ref, whh_n_ref,
              h_ref, mask_ref, valid_scr):
    h_ref[0:_H, :] = jnp.zeros((_H, _B), _F32)
    h_ref[_H:_H + 1, :] = jnp.ones((1, _B), _F32)
    valid_scr[...] = jnp.ones((1, _B), _F32)

    def step(t, carry):
        x = seq_ref[t]
        nz = (jnp.sum(x.astype(_F32), axis=0, keepdims=True) - 1.0) != 0.0
        v = jnp.logical_and(valid_scr[...] > 0.0, nz)
        valid_scr[...] = v.astype(_F32)

        @pl.when(t == 0)
        def _():
            mask_ref[...] = nz.astype(jnp.int32)

        h = h_ref[...]
        hb = h.astype(_BF16)
        r = jax.nn.sigmoid(
            jnp.dot(wih_r_ref[...], x, preferred_element_type=_F32)
            + jnp.dot(whh_r_ref[...], hb, preferred_element_type=_F32))
        n = jnp.tanh(
            jnp.dot(wih_n_ref[...], x, preferred_element_type=_F32)
            + r * jnp.dot(whh_n_ref[...], hb, preferred_element_type=_F32))
        z = jax.nn.sigmoid(
            jnp.dot(wih_z_ref[...], x, preferred_element_type=_F32)
            + jnp.dot(whh_z_ref[...], hb, preferred_element_type=_F32))
        hs = h[:_H, :]
        h_ref[0:_H, :] = jnp.where(v, n + z * (hs - n), hs)
        return carry

    jax.lax.fori_loop(0, _T, step, 0)


def _head_body(cards_ref, h_ref, mask_ref,
               w1_ref, w2_ref, wdp_ref, y_ref, xh_scr):
    xh_scr[0:_H, :] = jnp.maximum(
        jnp.dot(w1_ref[...], cards_ref[...], preferred_element_type=_F32), 0.0)
    xh_scr[_H:_H + 1, :] = jnp.ones((1, _B), _F32)
    yh = jnp.dot(w2_ref[...], xh_scr[...].astype(_BF16),
                 preferred_element_type=_F32)
    xdp = jnp.dot(wdp_ref[...], h_ref[...].astype(_BF16),
                  preferred_element_type=_F32)
    y_ref[...] = jnp.where(mask_ref[...] > 0, 0.3 * yh + 0.7 * xdp, yh)


def _aug(w, b):
    return jnp.concatenate([w, b[:, None]], axis=1).astype(_BF16)


def kernel(cards, discard_pile, hand_fc1_w, hand_fc1_b, hand_fc2_w, hand_fc2_b,
           gru_w_ih, gru_w_hh, gru_b_ih, gru_b_hh, dp_fc1_w, dp_fc1_b):
    seq = jnp.transpose(discard_pile.reshape(_B, _T, _IN), (1, 2, 0))
    seq = jnp.concatenate(
        [seq, jnp.ones((_T, 1, _B), seq.dtype)], axis=1).astype(_BF16)

    wih_r = _aug(gru_w_ih[:_H], gru_b_ih[:_H] + gru_b_hh[:_H])
    wih_z = _aug(gru_w_ih[_H:2 * _H],
                 gru_b_ih[_H:2 * _H] + gru_b_hh[_H:2 * _H])
    wih_n = _aug(gru_w_ih[2 * _H:], gru_b_ih[2 * _H:])
    whh_r = _aug(gru_w_hh[:_H], jnp.zeros((_H,), _F32))
    whh_z = _aug(gru_w_hh[_H:2 * _H], jnp.zeros((_H,), _F32))
    whh_n = _aug(gru_w_hh[2 * _H:], gru_b_hh[2 * _H:])

    h_aug, mask = pl.pallas_call(
        _gru_body,
        out_shape=[
            jax.ShapeDtypeStruct((_H + 1, _B), _F32),
            jax.ShapeDtypeStruct((1, _B), jnp.int32),
        ],
        scratch_shapes=[pltpu.VMEM((1, _B), _F32)],
        compiler_params=pltpu.CompilerParams(
            vmem_limit_bytes=64 * 1024 * 1024),
    )(seq, wih_r, wih_z, wih_n, whh_r, whh_z, whh_n)

    cards_t = jnp.concatenate(
        [cards.reshape(_B, _HAND).T, jnp.ones((1, _B), cards.dtype)],
        axis=0).astype(_BF16)
    w1 = _aug(hand_fc1_w, hand_fc1_b)
    w2 = _aug(hand_fc2_w, hand_fc2_b)
    wdp = _aug(dp_fc1_w, dp_fc1_b)

    y_t = pl.pallas_call(
        _head_body,
        out_shape=jax.ShapeDtypeStruct((_NA, _B), _F32),
        scratch_shapes=[pltpu.VMEM((_H + 1, _B), _F32)],
    )(cards_t, h_aug, mask, w1, w2, wdp)
    return y_t.T


# speedup vs baseline: 1.0196x; 1.0196x over previous
import jax
import jax.numpy as jnp
from jax.experimental import pallas as pl
from jax.experimental.pallas import tpu as pltpu

_B = 256
_T = 52
_IN = 52
_NA = 13 * 4 * 13 * 2
_H = _NA * 2
_HAND = 13 * 4 * 13

_F32 = jnp.float32
_BF16 = jnp.bfloat16


def _gru_body(seq_ref, wih_r_ref, wih_z_ref, wih_n_ref,
              whh_r_ref, whh_z_ref, whh_n_ref,
              h_ref, mask_ref, valid_scr):
    h_ref[0:_H, :] = jnp.zeros((_H, _B), _F32)
    h_ref[_H:_H + 1, :] = jnp.ones((1, _B), _F32)
    valid_scr[...] = jnp.ones((1, _B), _F32)

    def step(t, carry):
        x = seq_ref[t]
        nz = (jnp.sum(x.astype(_F32), axis=0, keepdims=True) - 1.0) != 0.0
        v = jnp.logical_and(valid_scr[...] > 0.0, nz)
        valid_scr[...] = v.astype(_F32)

        @pl.when(t == 0)
        def _():
            mask_ref[...] = nz.astype(jnp.int32)

        h = h_ref[...]
        hb = h.astype(_BF16)
        hs_b = hb[:_H, :]
        r = jax.nn.sigmoid(
            jnp.dot(wih_r_ref[...], x, preferred_element_type=_F32)
            + jnp.dot(whh_r_ref[...], hs_b, preferred_element_type=_F32))
        n = jnp.tanh(
            jnp.dot(wih_n_ref[...], x, preferred_element_type=_F32)
            + r * jnp.dot(whh_n_ref[...], hb, preferred_element_type=_F32))
        z = jax.nn.sigmoid(
            jnp.dot(wih_z_ref[...], x, preferred_element_type=_F32)
            + jnp.dot(whh_z_ref[...], hs_b, preferred_element_type=_F32))
        hs = h[:_H, :]
        h_ref[0:_H, :] = jnp.where(v, n + z * (hs - n), hs)
        return carry

    jax.lax.fori_loop(0, _T, step, 0)


def _head_body(cards_ref, h_ref, mask_ref,
               w1_ref, w2_ref, wdp_ref, y_ref, xh_scr):
    xh_scr[0:_H, :] = jnp.maximum(
        jnp.dot(w1_ref[...], cards_ref[...], preferred_element_type=_F32), 0.0)
    xh_scr[_H:_H + 1, :] = jnp.ones((1, _B), _F32)
    yh = jnp.dot(w2_ref[...], xh_scr[...].astype(_BF16),
                 preferred_element_type=_F32)
    xdp = jnp.dot(wdp_ref[...], h_ref[...].astype(_BF16),
                  preferred_element_type=_F32)
    y_ref[...] = jnp.where(mask_ref[...] > 0, 0.3 * yh + 0.7 * xdp, yh)


def _aug(w, b):
    return jnp.concatenate([w, b[:, None]], axis=1).astype(_BF16)


def kernel(cards, discard_pile, hand_fc1_w, hand_fc1_b, hand_fc2_w, hand_fc2_b,
           gru_w_ih, gru_w_hh, gru_b_ih, gru_b_hh, dp_fc1_w, dp_fc1_b):
    seq = jnp.transpose(discard_pile.reshape(_B, _T, _IN), (1, 2, 0))
    seq = jnp.concatenate(
        [seq, jnp.ones((_T, 1, _B), seq.dtype)], axis=1).astype(_BF16)

    wih_r = _aug(gru_w_ih[:_H], gru_b_ih[:_H] + gru_b_hh[:_H])
    wih_z = _aug(gru_w_ih[_H:2 * _H],
                 gru_b_ih[_H:2 * _H] + gru_b_hh[_H:2 * _H])
    wih_n = _aug(gru_w_ih[2 * _H:], gru_b_ih[2 * _H:])
    whh_r = gru_w_hh[:_H].astype(_BF16)
    whh_z = gru_w_hh[_H:2 * _H].astype(_BF16)
    whh_n = _aug(gru_w_hh[2 * _H:], gru_b_hh[2 * _H:])

    h_aug, mask = pl.pallas_call(
        _gru_body,
        out_shape=[
            jax.ShapeDtypeStruct((_H + 1, _B), _F32),
            jax.ShapeDtypeStruct((1, _B), jnp.int32),
        ],
        scratch_shapes=[pltpu.VMEM((1, _B), _F32)],
        compiler_params=pltpu.CompilerParams(
            vmem_limit_bytes=64 * 1024 * 1024),
    )(seq, wih_r, wih_z, wih_n, whh_r, whh_z, whh_n)

    cards_t = jnp.concatenate(
        [cards.reshape(_B, _HAND).T, jnp.ones((1, _B), cards.dtype)],
        axis=0).astype(_BF16)
    w1 = _aug(hand_fc1_w, hand_fc1_b)
    w2 = _aug(hand_fc2_w, hand_fc2_b)
    wdp = _aug(dp_fc1_w, dp_fc1_b)

    y_t = pl.pallas_call(
        _head_body,
        out_shape=jax.ShapeDtypeStruct((_NA, _B), _F32),
        scratch_shapes=[pltpu.VMEM((_H + 1, _B), _F32)],
    )(cards_t, h_aug, mask, w1, w2, wdp)
    return y_t.T
